# initial kernel scaffold (unmeasured)
import jax
import jax.numpy as jnp
from jax import lax
from jax.experimental import pallas as pl
from jax.experimental.pallas import tpu as pltpu

N_DEV = 8


def kernel(x, router_W, route_idx, expert_W, shared_W):
    n_tok, d_model = x.shape
    n_le, _, d_hid = expert_W.shape
    n_exp = N_DEV * n_le

    my = lax.axis_index("i")

    scores = x @ router_W
    probs = jax.nn.softmax(scores, axis=-1)
    onehot = jax.nn.one_hot(route_idx[:, 0], n_exp, dtype=probs.dtype)
    coef = probs * onehot

    origins = (my - jnp.arange(N_DEV)) % N_DEV
    perm = (n_le * origins)[:, None] + jnp.arange(n_le)[None, :]
    coef_r = jnp.take(coef, perm.reshape(-1), axis=1)

    xb = x.astype(jnp.bfloat16)
    ewb = expert_W.astype(jnp.bfloat16)
    swb = shared_W.astype(jnp.bfloat16)

    def body(x_ref, c_ref, ew_ref, sw_ref, out_ref, comm_ref,
             send_sems, recv_sems):
        my_pos = lax.axis_index("i")
        left = lax.rem(my_pos - 1 + N_DEV, N_DEV)
        right = lax.rem(my_pos + 1, N_DEV)

        barrier_sem = pltpu.get_barrier_semaphore()
        for nbr in [left, right]:
            pl.semaphore_signal(
                barrier_sem, inc=1,
                device_id=(nbr,), device_id_type=pl.DeviceIdType.MESH,
            )
        pl.semaphore_wait(barrier_sem, 2)

        def accum_block(k, w_ref):
            acc = out_ref[...]
            for j in range(n_le):
                y = jnp.dot(x_ref[...], w_ref[j],
                            preferred_element_type=jnp.float32)
                c_col = c_ref[:, n_le * k + j:n_le * k + j + 1]
                acc = acc + c_col * y
            out_ref[...] = acc

        out_ref[...] = jnp.dot(x_ref[...], sw_ref[...],
                               preferred_element_type=jnp.float32)
        accum_block(0, ew_ref)

        for h in range(N_DEV - 1):
            src = ew_ref if h == 0 else comm_ref.at[h - 1]
            rdma = pltpu.make_async_remote_copy(
                src_ref=src,
                dst_ref=comm_ref.at[h],
                send_sem=send_sems.at[h],
                recv_sem=recv_sems.at[h],
                device_id=(right,),
                device_id_type=pl.DeviceIdType.MESH,
            )
            rdma.start()
            rdma.wait()
            accum_block(h + 1, comm_ref.at[h])

    return pl.pallas_call(
        body,
        out_shape=jax.ShapeDtypeStruct((n_tok, d_hid), jnp.float32),
        in_specs=[
            pl.BlockSpec(memory_space=pltpu.VMEM),
            pl.BlockSpec(memory_space=pltpu.VMEM),
            pl.BlockSpec(memory_space=pltpu.VMEM),
            pl.BlockSpec(memory_space=pltpu.VMEM),
        ],
        out_specs=pl.BlockSpec(memory_space=pltpu.VMEM),
        scratch_shapes=[
            pltpu.VMEM((N_DEV - 1, n_le, d_model, d_hid), jnp.bfloat16),
            pltpu.SemaphoreType.DMA((N_DEV - 1,)),
            pltpu.SemaphoreType.DMA((N_DEV - 1,)),
        ],
        compiler_params=pltpu.CompilerParams(collective_id=0),
    )(xb, coef_r, ewb, swb)


# baseline (device time: 373034 ns/iter reference)
import jax
import jax.numpy as jnp
from jax import lax
from jax.experimental import pallas as pl
from jax.experimental.pallas import tpu as pltpu

N_DEV = 8


def kernel(x, router_W, route_idx, expert_W, shared_W):
    n_tok, d_model = x.shape
    n_le, _, d_hid = expert_W.shape
    n_exp = N_DEV * n_le

    my = lax.axis_index("i")

    scores = x @ router_W
    probs = jax.nn.softmax(scores, axis=-1)
    onehot = jax.nn.one_hot(route_idx[:, 0], n_exp, dtype=probs.dtype)
    coef = probs * onehot

    origins = (my - jnp.arange(N_DEV)) % N_DEV
    perm = (n_le * origins)[:, None] + jnp.arange(n_le)[None, :]
    coef_r = jnp.take(coef, perm.reshape(-1), axis=1)
    coef_blk = coef_r.reshape(n_tok, N_DEV, n_le).transpose(1, 0, 2)
    coef_blk = coef_blk.astype(jnp.bfloat16)

    xb = x.astype(jnp.bfloat16)
    ewb = expert_W.astype(jnp.bfloat16)
    swb = shared_W.astype(jnp.bfloat16)

    def body(x_ref, c_ref, ew_ref, sw_ref, out_ref, comm_ref,
             send_sems, recv_sems):
        my_pos = lax.axis_index("i")
        left = lax.rem(my_pos - 1 + N_DEV, N_DEV)
        right = lax.rem(my_pos + 1, N_DEV)

        barrier_sem = pltpu.get_barrier_semaphore()
        for nbr in [left, right]:
            pl.semaphore_signal(
                barrier_sem, inc=1,
                device_id=(nbr,), device_id_type=pl.DeviceIdType.MESH,
            )
        pl.semaphore_wait(barrier_sem, 2)

        comm_ref[0] = ew_ref[...]
        out_ref[...] = jnp.dot(x_ref[...], sw_ref[...],
                               preferred_element_type=jnp.float32)

        def step(k, _):
            kk = lax.min(k, N_DEV - 2)
            rdma = pltpu.make_async_remote_copy(
                src_ref=comm_ref.at[kk],
                dst_ref=comm_ref.at[kk + 1],
                send_sem=send_sems.at[kk],
                recv_sem=recv_sems.at[kk],
                device_id=(right,),
                device_id_type=pl.DeviceIdType.MESH,
            )

            @pl.when(k < N_DEV - 1)
            def _():
                rdma.start()

            for j in range(n_le):
                y = jnp.dot(x_ref[...], comm_ref[k, j],
                            preferred_element_type=jnp.float32)
                out_ref[...] = out_ref[...] + c_ref[k, :, j:j + 1] * y

            @pl.when(k < N_DEV - 1)
            def _():
                rdma.wait()

            return 0

        lax.fori_loop(0, N_DEV, step, 0)

    return pl.pallas_call(
        body,
        out_shape=jax.ShapeDtypeStruct((n_tok, d_hid), jnp.float32),
        in_specs=[
            pl.BlockSpec(memory_space=pltpu.VMEM),
            pl.BlockSpec(memory_space=pltpu.VMEM),
            pl.BlockSpec(memory_space=pltpu.VMEM),
            pl.BlockSpec(memory_space=pltpu.VMEM),
        ],
        out_specs=pl.BlockSpec(memory_space=pltpu.VMEM),
        scratch_shapes=[
            pltpu.VMEM((N_DEV, n_le, d_model, d_hid), jnp.bfloat16),
            pltpu.SemaphoreType.DMA((N_DEV - 1,)),
            pltpu.SemaphoreType.DMA((N_DEV - 1,)),
        ],
        compiler_params=pltpu.CompilerParams(
            collective_id=0, vmem_limit_bytes=62 * 1024 * 1024,
        ),
    )(xb, coef_blk, ewb, swb)
